# baseline (speedup 1.0000x reference)
import functools

import jax
import jax.numpy as jnp
from jax import lax
from jax.experimental import pallas as pl
from jax.experimental.pallas import tpu as pltpu
from jax.experimental.pallas import tpu_sc as plsc

N = 10000
E = 160000
DF = 128
DE = 4
DS = 8
OX = 128
OE = 16

NC = 2
NS = 16
NW = NC * NS

EPW = E // NW
CHUNK = 125
NCHUNK = EPW // CHUNK

NPAD = 10240
RPT = NPAD // NS


def _sc_segment_sum(idx_g, attr_g, zeros_t):
    mesh = plsc.VectorSubcoreMesh(
        core_axis_name="c", subcore_axis_name="s", num_cores=NC, num_subcores=NS
    )

    @functools.partial(
        pl.kernel,
        out_type=jax.ShapeDtypeStruct((NC, NPAD, DS), jnp.float32),
        mesh=mesh,
        scratch_types=[
            pltpu.VMEM((NCHUNK, CHUNK), jnp.int32),
            pltpu.VMEM((NCHUNK, CHUNK, DS), jnp.float32),
            pltpu.VMEM((RPT, DS), jnp.float32),
            pltpu.VMEM_SHARED((NPAD, DS), jnp.float32),
        ],
        compiler_params=pltpu.CompilerParams(use_tc_tiling_on_sc=False),
    )
    def sc_kernel(idx_hbm, attr_hbm, zeros_hbm, out_hbm, idx_v, attr_v, stage_v, adj_sh):
        cid = lax.axis_index("c")
        sid = lax.axis_index("s")
        wid = cid * NS + sid

        rows = pl.ds(sid * RPT, RPT)
        pltpu.sync_copy(zeros_hbm, stage_v)
        pltpu.sync_copy(stage_v, adj_sh.at[rows])

        pltpu.sync_copy(idx_hbm.at[wid], idx_v)
        pltpu.sync_copy(attr_hbm.at[wid], attr_v.at[:, :, pl.ds(0, DE)])

        plsc.subcore_barrier()

        @pl.loop(0, NCHUNK)
        def _(j):
            pltpu.sync_copy(attr_v.at[j], adj_sh.at[idx_v.at[j]], add=True)

        plsc.subcore_barrier()

        pltpu.sync_copy(adj_sh.at[rows], stage_v)
        pltpu.sync_copy(stage_v, out_hbm.at[cid, rows])

    return sc_kernel(idx_g, attr_g, zeros_t)


def _xl_body(x_ref, wx_ref, bx_ref, o_ref):
    xl = lax.dot_general(
        x_ref[...], wx_ref[...], (((1,), (1,)), ((), ())),
        preferred_element_type=jnp.float32, precision=lax.Precision.HIGHEST,
    ) + bx_ref[...]
    blk = x_ref.shape[0]
    o_ref[...] = jnp.concatenate(
        [jnp.maximum(xl, 0.0), jnp.zeros((blk, OE), jnp.float32)], axis=1
    )


def _xl_call(x, Wx, bx2):
    blk = 2000
    return pl.pallas_call(
        _xl_body,
        grid=(N // blk,),
        in_specs=[
            pl.BlockSpec((blk, DF), lambda i: (i, 0)),
            pl.BlockSpec((OX, DF), lambda i: (0, 0)),
            pl.BlockSpec((1, OX), lambda i: (0, 0)),
        ],
        out_specs=pl.BlockSpec((blk, OX + OE), lambda i: (i, 0)),
        out_shape=jax.ShapeDtypeStruct((N, OX + OE), jnp.float32),
    )(x, Wx, bx2)


def _el_body(adj_ref, we_ref, be_ref, o1_ref, o_ref):
    adj = adj_ref[0, :, 0:DE]
    for c in range(1, NC):
        adj = adj + adj_ref[c, :, 0:DE]
    el = lax.dot_general(
        adj, we_ref[...], (((1,), (1,)), ((), ())),
        preferred_element_type=jnp.float32, precision=lax.Precision.HIGHEST,
    ) + be_ref[...]
    o_ref[...] = jnp.concatenate(
        [o1_ref[:, 0:OX], jnp.maximum(el, 0.0)], axis=1
    )


def _el_call(adj2, We, be2, out1):
    blk = 2000
    return pl.pallas_call(
        _el_body,
        grid=(N // blk,),
        in_specs=[
            pl.BlockSpec((NC, blk, DS), lambda i: (0, i, 0)),
            pl.BlockSpec((OE, DE), lambda i: (0, 0)),
            pl.BlockSpec((1, OE), lambda i: (0, 0)),
            pl.BlockSpec((blk, OX + OE), lambda i: (i, 0)),
        ],
        out_specs=pl.BlockSpec((blk, OX + OE), lambda i: (i, 0)),
        out_shape=jax.ShapeDtypeStruct((N, OX + OE), jnp.float32),
        input_output_aliases={3: 0},
    )(adj2, We, be2, out1)


def kernel(x, edge_index, edge_attr, Wx, bx, We, be):
    src = edge_index[0]
    idx_g = src.reshape(NW, NCHUNK, CHUNK)
    attr_g = edge_attr.reshape(NW, NCHUNK, CHUNK, DE)
    zeros_t = jnp.zeros((RPT, DS), jnp.float32)
    adj2 = _sc_segment_sum(idx_g, attr_g, zeros_t)
    return _fuse_call(x, Wx, bx.reshape(1, OX), adj2, We, be.reshape(1, OE))


def _fuse_body(x_ref, wx_ref, bx_ref, adj_ref, we_ref, be_ref, o_ref):
    xl = lax.dot_general(
        x_ref[...], wx_ref[...], (((1,), (1,)), ((), ())),
        preferred_element_type=jnp.float32, precision=lax.Precision.HIGHEST,
    ) + bx_ref[...]
    adj = adj_ref[0, :, 0:DE]
    for c in range(1, NC):
        adj = adj + adj_ref[c, :, 0:DE]
    el = lax.dot_general(
        adj, we_ref[...], (((1,), (1,)), ((), ())),
        preferred_element_type=jnp.float32, precision=lax.Precision.HIGHEST,
    ) + be_ref[...]
    o_ref[...] = jnp.maximum(jnp.concatenate([xl, el], axis=1), 0.0)


def _fuse_call(x, Wx, bx2, adj2, We, be2):
    blk = 2000
    return pl.pallas_call(
        _fuse_body,
        grid=(N // blk,),
        in_specs=[
            pl.BlockSpec((blk, DF), lambda i: (i, 0)),
            pl.BlockSpec((OX, DF), lambda i: (0, 0)),
            pl.BlockSpec((1, OX), lambda i: (0, 0)),
            pl.BlockSpec((NC, blk, DS), lambda i: (0, i, 0)),
            pl.BlockSpec((OE, DE), lambda i: (0, 0)),
            pl.BlockSpec((1, OE), lambda i: (0, 0)),
        ],
        out_specs=pl.BlockSpec((blk, OX + OE), lambda i: (i, 0)),
        out_shape=jax.ShapeDtypeStruct((N, OX + OE), jnp.float32),
    )(x, Wx, bx2, adj2, We, be2)


# speedup vs baseline: 1.3488x; 1.3488x over previous
import functools

import jax
import jax.numpy as jnp
from jax import lax
from jax.experimental import pallas as pl
from jax.experimental.pallas import tpu as pltpu
from jax.experimental.pallas import tpu_sc as plsc

N = 10000
E = 160000
DF = 128
DE = 4
DS = 8
OX = 128
OE = 16

NC = 2
NS = 16
NW = NC * NS

EPW = E // NW
CHUNK = 125
NCHUNK = EPW // CHUNK

NPAD = 10240
RPT = NPAD // NS


def _sc_segment_sum(idx_g, attr_g, zeros_t):
    mesh = plsc.VectorSubcoreMesh(
        core_axis_name="c", subcore_axis_name="s", num_cores=NC, num_subcores=NS
    )

    @functools.partial(
        pl.kernel,
        out_type=jax.ShapeDtypeStruct((NC, NPAD, DS), jnp.float32),
        mesh=mesh,
        scratch_types=[
            pltpu.VMEM((NCHUNK, CHUNK), jnp.int32),
            pltpu.VMEM((NCHUNK, CHUNK, DS), jnp.float32),
            pltpu.VMEM((RPT, DS), jnp.float32),
            pltpu.VMEM_SHARED((NPAD, DS), jnp.float32),
        ],
        compiler_params=pltpu.CompilerParams(use_tc_tiling_on_sc=False),
    )
    def sc_kernel(idx_hbm, attr_hbm, zeros_hbm, out_hbm, idx_v, attr_v, stage_v, adj_sh):
        cid = lax.axis_index("c")
        sid = lax.axis_index("s")
        wid = cid * NS + sid

        rows = pl.ds(sid * RPT, RPT)
        pltpu.sync_copy(zeros_hbm, stage_v)
        pltpu.sync_copy(stage_v, adj_sh.at[rows])

        pltpu.sync_copy(idx_hbm.at[wid], idx_v)
        pltpu.sync_copy(attr_hbm.at[wid], attr_v)

        plsc.subcore_barrier()

        @pl.loop(0, NCHUNK)
        def _(j):
            pltpu.sync_copy(attr_v.at[j], adj_sh.at[idx_v.at[j]], add=True)

        plsc.subcore_barrier()

        pltpu.sync_copy(adj_sh.at[rows], stage_v)
        pltpu.sync_copy(stage_v, out_hbm.at[cid, rows])

    return sc_kernel(idx_g, attr_g, zeros_t)


def _xl_body(x_ref, wx_ref, bx_ref, o_ref):
    xl = lax.dot_general(
        x_ref[...], wx_ref[...], (((1,), (1,)), ((), ())),
        preferred_element_type=jnp.float32, precision=lax.Precision.HIGHEST,
    ) + bx_ref[...]
    blk = x_ref.shape[0]
    o_ref[...] = jnp.concatenate(
        [jnp.maximum(xl, 0.0), jnp.zeros((blk, OE), jnp.float32)], axis=1
    )


def _xl_call(x, Wx, bx2):
    blk = 2000
    return pl.pallas_call(
        _xl_body,
        grid=(N // blk,),
        in_specs=[
            pl.BlockSpec((blk, DF), lambda i: (i, 0)),
            pl.BlockSpec((OX, DF), lambda i: (0, 0)),
            pl.BlockSpec((1, OX), lambda i: (0, 0)),
        ],
        out_specs=pl.BlockSpec((blk, OX + OE), lambda i: (i, 0)),
        out_shape=jax.ShapeDtypeStruct((N, OX + OE), jnp.float32),
    )(x, Wx, bx2)


def _el_body(adj_ref, we_ref, be_ref, o1_ref, o_ref):
    adj = adj_ref[0, :, 0:DE]
    for c in range(1, NC):
        adj = adj + adj_ref[c, :, 0:DE]
    el = lax.dot_general(
        adj, we_ref[...], (((1,), (1,)), ((), ())),
        preferred_element_type=jnp.float32, precision=lax.Precision.HIGHEST,
    ) + be_ref[...]
    o_ref[...] = jnp.concatenate(
        [o1_ref[:, 0:OX], jnp.maximum(el, 0.0)], axis=1
    )


def _el_call(adj2, We, be2, out1):
    blk = 2000
    return pl.pallas_call(
        _el_body,
        grid=(N // blk,),
        in_specs=[
            pl.BlockSpec((NC, blk, DS), lambda i: (0, i, 0)),
            pl.BlockSpec((OE, DE), lambda i: (0, 0)),
            pl.BlockSpec((1, OE), lambda i: (0, 0)),
            pl.BlockSpec((blk, OX + OE), lambda i: (i, 0)),
        ],
        out_specs=pl.BlockSpec((blk, OX + OE), lambda i: (i, 0)),
        out_shape=jax.ShapeDtypeStruct((N, OX + OE), jnp.float32),
        input_output_aliases={3: 0},
    )(adj2, We, be2, out1)


def kernel(x, edge_index, edge_attr, Wx, bx, We, be):
    src = edge_index[0]
    idx_g = src.reshape(NW, NCHUNK, CHUNK)
    attr_g = jnp.pad(edge_attr, ((0, 0), (0, DS - DE))).reshape(NW, NCHUNK, CHUNK, DS)
    zeros_t = jnp.zeros((RPT, DS), jnp.float32)
    adj2 = _sc_segment_sum(idx_g, attr_g, zeros_t)
    out1 = _xl_call(x, Wx, bx.reshape(1, OX))
    return _el_call(adj2, We, be.reshape(1, OE), out1)


def _fuse_body(x_ref, wx_ref, bx_ref, adj_ref, we_ref, be_ref, o_ref):
    xl = lax.dot_general(
        x_ref[...], wx_ref[...], (((1,), (1,)), ((), ())),
        preferred_element_type=jnp.float32, precision=lax.Precision.HIGHEST,
    ) + bx_ref[...]
    adj = adj_ref[0, :, 0:DE]
    for c in range(1, NC):
        adj = adj + adj_ref[c, :, 0:DE]
    el = lax.dot_general(
        adj, we_ref[...], (((1,), (1,)), ((), ())),
        preferred_element_type=jnp.float32, precision=lax.Precision.HIGHEST,
    ) + be_ref[...]
    o_ref[...] = jnp.maximum(jnp.concatenate([xl, el], axis=1), 0.0)


def _fuse_call(x, Wx, bx2, adj2, We, be2):
    blk = 2000
    return pl.pallas_call(
        _fuse_body,
        grid=(N // blk,),
        in_specs=[
            pl.BlockSpec((blk, DF), lambda i: (i, 0)),
            pl.BlockSpec((OX, DF), lambda i: (0, 0)),
            pl.BlockSpec((1, OX), lambda i: (0, 0)),
            pl.BlockSpec((NC, blk, DS), lambda i: (0, i, 0)),
            pl.BlockSpec((OE, DE), lambda i: (0, 0)),
            pl.BlockSpec((1, OE), lambda i: (0, 0)),
        ],
        out_specs=pl.BlockSpec((blk, OX + OE), lambda i: (i, 0)),
        out_shape=jax.ShapeDtypeStruct((N, OX + OE), jnp.float32),
    )(x, Wx, bx2, adj2, We, be2)


# speedup vs baseline: 1.3499x; 1.0009x over previous
import functools

import jax
import jax.numpy as jnp
from jax import lax
from jax.experimental import pallas as pl
from jax.experimental.pallas import tpu as pltpu
from jax.experimental.pallas import tpu_sc as plsc

N = 10000
E = 160000
DF = 128
DE = 4
DS = 8
OX = 128
OE = 16

NC = 2
NS = 16
NW = NC * NS

EPW = E // NW
CHUNK = 125
NCHUNK = EPW // CHUNK

NPAD = 10240
RPT = NPAD // NS


def _sc_segment_sum(idx_g, attr_g, zeros_t):
    mesh = plsc.VectorSubcoreMesh(
        core_axis_name="c", subcore_axis_name="s", num_cores=NC, num_subcores=NS
    )

    @functools.partial(
        pl.kernel,
        out_type=jax.ShapeDtypeStruct((NC, NPAD, DS), jnp.float32),
        mesh=mesh,
        scratch_types=[
            pltpu.VMEM((NCHUNK, CHUNK), jnp.int32),
            pltpu.VMEM((NCHUNK, CHUNK, DS), jnp.float32),
            pltpu.VMEM((RPT, DS), jnp.float32),
            pltpu.VMEM_SHARED((NPAD, DS), jnp.float32),
        ],
        compiler_params=pltpu.CompilerParams(use_tc_tiling_on_sc=False),
    )
    def sc_kernel(idx_hbm, attr_hbm, zeros_hbm, out_hbm, idx_v, attr_v, stage_v, adj_sh):
        cid = lax.axis_index("c")
        sid = lax.axis_index("s")
        wid = cid * NS + sid

        rows = pl.ds(sid * RPT, RPT)
        pltpu.sync_copy(zeros_hbm, stage_v)
        pltpu.sync_copy(stage_v, adj_sh.at[rows])

        pltpu.sync_copy(idx_hbm.at[wid], idx_v)
        pltpu.sync_copy(attr_hbm.at[wid], attr_v)

        plsc.subcore_barrier()

        @pl.loop(0, NCHUNK)
        def _(j):
            pltpu.sync_copy(attr_v.at[j], adj_sh.at[idx_v.at[j]], add=True)

        plsc.subcore_barrier()

        pltpu.sync_copy(adj_sh.at[rows], stage_v)
        pltpu.sync_copy(stage_v, out_hbm.at[cid, rows])

    return sc_kernel(idx_g, attr_g, zeros_t)


def _xl_body(x_ref, wx_ref, bx_ref, o_ref):
    xl = lax.dot_general(
        x_ref[...], wx_ref[...], (((1,), (1,)), ((), ())),
        preferred_element_type=jnp.float32, precision=lax.Precision.HIGHEST,
    ) + bx_ref[...]
    blk = x_ref.shape[0]
    o_ref[...] = jnp.concatenate(
        [jnp.maximum(xl, 0.0), jnp.zeros((blk, OE), jnp.float32)], axis=1
    )


def _xl_call(x, Wx, bx2):
    blk = 2000
    return pl.pallas_call(
        _xl_body,
        grid=(N // blk,),
        in_specs=[
            pl.BlockSpec((blk, DF), lambda i: (i, 0)),
            pl.BlockSpec((OX, DF), lambda i: (0, 0)),
            pl.BlockSpec((1, OX), lambda i: (0, 0)),
        ],
        out_specs=pl.BlockSpec((blk, OX + OE), lambda i: (i, 0)),
        out_shape=jax.ShapeDtypeStruct((N, OX + OE), jnp.float32),
    )(x, Wx, bx2)


def _el_body(adj_ref, we_ref, be_ref, o1_ref, o_ref):
    adj = adj_ref[0, :, 0:DE]
    for c in range(1, NC):
        adj = adj + adj_ref[c, :, 0:DE]
    el = lax.dot_general(
        adj, we_ref[...], (((1,), (1,)), ((), ())),
        preferred_element_type=jnp.float32, precision=lax.Precision.HIGHEST,
    ) + be_ref[...]
    o_ref[...] = jnp.concatenate(
        [o1_ref[:, 0:OX], jnp.maximum(el, 0.0)], axis=1
    )


def _el_call(adj2, We, be2, out1):
    blk = 2000
    return pl.pallas_call(
        _el_body,
        grid=(N // blk,),
        in_specs=[
            pl.BlockSpec((NC, blk, DS), lambda i: (0, i, 0)),
            pl.BlockSpec((OE, DE), lambda i: (0, 0)),
            pl.BlockSpec((1, OE), lambda i: (0, 0)),
            pl.BlockSpec((blk, OX + OE), lambda i: (i, 0)),
        ],
        out_specs=pl.BlockSpec((blk, OX + OE), lambda i: (i, 0)),
        out_shape=jax.ShapeDtypeStruct((N, OX + OE), jnp.float32),
        input_output_aliases={3: 0},
    )(adj2, We, be2, out1)


def kernel(x, edge_index, edge_attr, Wx, bx, We, be):
    src = edge_index[0]
    idx_g = src.reshape(NW, NCHUNK, CHUNK)
    attr_g = jnp.pad(edge_attr, ((0, 0), (0, DS - DE))).reshape(NW, NCHUNK, CHUNK, DS)
    zeros_t = jnp.zeros((RPT, DS), jnp.float32)
    out1 = _xl_call(x, Wx, bx.reshape(1, OX))
    adj2 = _sc_segment_sum(idx_g, attr_g, zeros_t)
    return _el_call(adj2, We, be.reshape(1, OE), out1)


def _fuse_body(x_ref, wx_ref, bx_ref, adj_ref, we_ref, be_ref, o_ref):
    xl = lax.dot_general(
        x_ref[...], wx_ref[...], (((1,), (1,)), ((), ())),
        preferred_element_type=jnp.float32, precision=lax.Precision.HIGHEST,
    ) + bx_ref[...]
    adj = adj_ref[0, :, 0:DE]
    for c in range(1, NC):
        adj = adj + adj_ref[c, :, 0:DE]
    el = lax.dot_general(
        adj, we_ref[...], (((1,), (1,)), ((), ())),
        preferred_element_type=jnp.float32, precision=lax.Precision.HIGHEST,
    ) + be_ref[...]
    o_ref[...] = jnp.maximum(jnp.concatenate([xl, el], axis=1), 0.0)


def _fuse_call(x, Wx, bx2, adj2, We, be2):
    blk = 2000
    return pl.pallas_call(
        _fuse_body,
        grid=(N // blk,),
        in_specs=[
            pl.BlockSpec((blk, DF), lambda i: (i, 0)),
            pl.BlockSpec((OX, DF), lambda i: (0, 0)),
            pl.BlockSpec((1, OX), lambda i: (0, 0)),
            pl.BlockSpec((NC, blk, DS), lambda i: (0, i, 0)),
            pl.BlockSpec((OE, DE), lambda i: (0, 0)),
            pl.BlockSpec((1, OE), lambda i: (0, 0)),
        ],
        out_specs=pl.BlockSpec((blk, OX + OE), lambda i: (i, 0)),
        out_shape=jax.ShapeDtypeStruct((N, OX + OE), jnp.float32),
    )(x, Wx, bx2, adj2, We, be2)


# speedup vs baseline: 1.3571x; 1.0053x over previous
import functools

import jax
import jax.numpy as jnp
from jax import lax
from jax.experimental import pallas as pl
from jax.experimental.pallas import tpu as pltpu
from jax.experimental.pallas import tpu_sc as plsc

N = 10000
E = 160000
DF = 128
DE = 4
DS = 8
OX = 128
OE = 16

NC = 2
NS = 16
NW = NC * NS

EPW = E // NW
CHUNK = 125
NCHUNK = EPW // CHUNK

NPAD = 10240
RPT = NPAD // NS


def _sc_segment_sum(idx_g, attr_g, zeros_t):
    mesh = plsc.VectorSubcoreMesh(
        core_axis_name="c", subcore_axis_name="s", num_cores=NC, num_subcores=NS
    )

    @functools.partial(
        pl.kernel,
        out_type=jax.ShapeDtypeStruct((NC, NPAD, DS), jnp.float32),
        mesh=mesh,
        scratch_types=[
            pltpu.VMEM((NCHUNK, CHUNK), jnp.int32),
            pltpu.VMEM((NCHUNK, CHUNK, DS), jnp.float32),
            pltpu.VMEM((RPT, DS), jnp.float32),
            pltpu.VMEM_SHARED((NPAD, DS), jnp.float32),
        ],
        compiler_params=pltpu.CompilerParams(use_tc_tiling_on_sc=False),
    )
    def sc_kernel(idx_hbm, attr_hbm, zeros_hbm, out_hbm, idx_v, attr_v, stage_v, adj_sh):
        cid = lax.axis_index("c")
        sid = lax.axis_index("s")
        wid = cid * NS + sid

        rows = pl.ds(sid * RPT, RPT)
        pltpu.sync_copy(zeros_hbm, stage_v)
        pltpu.sync_copy(stage_v, adj_sh.at[rows])

        pltpu.sync_copy(idx_hbm.at[wid], idx_v)
        pltpu.sync_copy(attr_hbm.at[wid], attr_v)

        plsc.subcore_barrier()

        @pl.loop(0, NCHUNK)
        def _(j):
            pltpu.sync_copy(attr_v.at[j], adj_sh.at[idx_v.at[j]], add=True)

        plsc.subcore_barrier()

        pltpu.sync_copy(adj_sh.at[rows], stage_v)
        pltpu.sync_copy(stage_v, out_hbm.at[cid, rows])

    return sc_kernel(idx_g, attr_g, zeros_t)


def _xl_body(x_ref, wx_ref, bx_ref, o_ref):
    xl = lax.dot_general(
        x_ref[...], wx_ref[...], (((1,), (1,)), ((), ())),
        preferred_element_type=jnp.float32,
    ) + bx_ref[...]
    blk = x_ref.shape[0]
    o_ref[...] = jnp.concatenate(
        [jnp.maximum(xl, 0.0), jnp.zeros((blk, OE), jnp.float32)], axis=1
    )


def _xl_call(x, Wx, bx2):
    blk = 2000
    return pl.pallas_call(
        _xl_body,
        grid=(N // blk,),
        in_specs=[
            pl.BlockSpec((blk, DF), lambda i: (i, 0)),
            pl.BlockSpec((OX, DF), lambda i: (0, 0)),
            pl.BlockSpec((1, OX), lambda i: (0, 0)),
        ],
        out_specs=pl.BlockSpec((blk, OX + OE), lambda i: (i, 0)),
        out_shape=jax.ShapeDtypeStruct((N, OX + OE), jnp.float32),
    )(x, Wx, bx2)


def _el_body(adj_ref, we_ref, be_ref, o1_ref, o_ref):
    adj = adj_ref[0, :, 0:DE]
    for c in range(1, NC):
        adj = adj + adj_ref[c, :, 0:DE]
    el = lax.dot_general(
        adj, we_ref[...], (((1,), (1,)), ((), ())),
        preferred_element_type=jnp.float32,
    ) + be_ref[...]
    o_ref[...] = jnp.concatenate(
        [o1_ref[:, 0:OX], jnp.maximum(el, 0.0)], axis=1
    )


def _el_call(adj2, We, be2, out1):
    blk = 2000
    return pl.pallas_call(
        _el_body,
        grid=(N // blk,),
        in_specs=[
            pl.BlockSpec((NC, blk, DS), lambda i: (0, i, 0)),
            pl.BlockSpec((OE, DE), lambda i: (0, 0)),
            pl.BlockSpec((1, OE), lambda i: (0, 0)),
            pl.BlockSpec((blk, OX + OE), lambda i: (i, 0)),
        ],
        out_specs=pl.BlockSpec((blk, OX + OE), lambda i: (i, 0)),
        out_shape=jax.ShapeDtypeStruct((N, OX + OE), jnp.float32),
        input_output_aliases={3: 0},
    )(adj2, We, be2, out1)


def kernel(x, edge_index, edge_attr, Wx, bx, We, be):
    src = edge_index[0]
    idx_g = src.reshape(NW, NCHUNK, CHUNK)
    attr_g = jnp.pad(edge_attr, ((0, 0), (0, DS - DE))).reshape(NW, NCHUNK, CHUNK, DS)
    zeros_t = jnp.zeros((RPT, DS), jnp.float32)
    out1 = _xl_call(x, Wx, bx.reshape(1, OX))
    adj2 = _sc_segment_sum(idx_g, attr_g, zeros_t)
    return _el_call(adj2, We, be.reshape(1, OE), out1)


def _fuse_body(x_ref, wx_ref, bx_ref, adj_ref, we_ref, be_ref, o_ref):
    xl = lax.dot_general(
        x_ref[...], wx_ref[...], (((1,), (1,)), ((), ())),
        preferred_element_type=jnp.float32,
    ) + bx_ref[...]
    adj = adj_ref[0, :, 0:DE]
    for c in range(1, NC):
        adj = adj + adj_ref[c, :, 0:DE]
    el = lax.dot_general(
        adj, we_ref[...], (((1,), (1,)), ((), ())),
        preferred_element_type=jnp.float32,
    ) + be_ref[...]
    o_ref[...] = jnp.maximum(jnp.concatenate([xl, el], axis=1), 0.0)


def _fuse_call(x, Wx, bx2, adj2, We, be2):
    blk = 2000
    return pl.pallas_call(
        _fuse_body,
        grid=(N // blk,),
        in_specs=[
            pl.BlockSpec((blk, DF), lambda i: (i, 0)),
            pl.BlockSpec((OX, DF), lambda i: (0, 0)),
            pl.BlockSpec((1, OX), lambda i: (0, 0)),
            pl.BlockSpec((NC, blk, DS), lambda i: (0, i, 0)),
            pl.BlockSpec((OE, DE), lambda i: (0, 0)),
            pl.BlockSpec((1, OE), lambda i: (0, 0)),
        ],
        out_specs=pl.BlockSpec((blk, OX + OE), lambda i: (i, 0)),
        out_shape=jax.ShapeDtypeStruct((N, OX + OE), jnp.float32),
    )(x, Wx, bx2, adj2, We, be2)
